# baseline (speedup 1.0000x reference)
import functools

import jax
import jax.numpy as jnp
from jax.experimental import pallas as pl


def _se_kernel(x_ref, w1t_ref, w2t_ref, y_ref, *, inv_n):
    hi = pl.program_id(1)
    nh = pl.num_programs(1)
    part = jnp.sum(x_ref[0], axis=(1, 2))[None, :]

    @pl.when(hi == 0)
    def _():
        y_ref[0] = part

    @pl.when(hi > 0)
    def _():
        y_ref[0] += part

    @pl.when(hi == nh - 1)
    def _():
        m = y_ref[0] * inv_n
        t = jnp.maximum(
            jnp.dot(m, w1t_ref[...], preferred_element_type=jnp.float32), 0.0)
        y_ref[0] = jax.nn.sigmoid(
            jnp.dot(t, w2t_ref[...], preferred_element_type=jnp.float32))


def _gcn_kernel(x_ref, xh_ref, y_ref, wg_ref, bg_ref, o_ref, *, hb, w):
    hi = pl.program_id(1)
    c = x_ref.shape[1]
    m = hb * w

    y = y_ref[0, 0]
    wgy = wg_ref[...] * y[None, :]
    zw = jnp.dot(wgy, x_ref[0], preferred_element_type=jnp.float32)

    ri = jax.lax.broadcasted_iota(jnp.int32, (1, hb, w), 1) + hi * hb
    ci = jax.lax.broadcasted_iota(jnp.int32, (1, hb, w), 2)
    dinv = jax.lax.rsqrt(1.0 + (ri > 0).astype(jnp.float32)
                         + (ci > 0).astype(jnp.float32)).reshape(1, m)
    maskj = (ci > 0).astype(jnp.float32).reshape(1, m)
    z = zw * dinv

    zh = jnp.dot(wgy, xh_ref[0, 0], preferred_element_type=jnp.float32)
    cj = jax.lax.broadcasted_iota(jnp.int32, (1, w), 1)
    dinv_h = jax.lax.rsqrt(2.0 + (cj > 0).astype(jnp.float32))
    zh = jnp.where(hi == 0, 0.0, zh * dinv_h)

    zd = jnp.concatenate([zh, z[:, :m - w]], axis=1)
    zr = jnp.roll(z, 1, axis=1) * maskj

    o_ref[0] = (z + zd + zr) * dinv + bg_ref[0][:, None]


@jax.jit
def kernel(x, W1, W2, Wg, bg):
    b, c, h, w = x.shape
    hb = 64
    nh = h // hb
    n = h * w

    y = pl.pallas_call(
        functools.partial(_se_kernel, inv_n=1.0 / n),
        grid=(b, nh),
        in_specs=[
            pl.BlockSpec((1, c, hb, w), lambda bi, hi: (bi, 0, hi, 0)),
            pl.BlockSpec((c, W1.shape[0]), lambda bi, hi: (0, 0)),
            pl.BlockSpec((W1.shape[0], c), lambda bi, hi: (0, 0)),
        ],
        out_specs=pl.BlockSpec((1, 1, c), lambda bi, hi: (bi, 0, 0)),
        out_shape=jax.ShapeDtypeStruct((b, 1, c), jnp.float32),
    )(x, W1.T, W2.T)

    halo = jnp.concatenate(
        [jnp.zeros((b, c, 1, w), x.dtype), x[:, :, hb - 1:h - 1:hb, :]],
        axis=2).transpose(0, 2, 1, 3)

    out = pl.pallas_call(
        functools.partial(_gcn_kernel, hb=hb, w=w),
        grid=(b, nh),
        in_specs=[
            pl.BlockSpec((1, c, hb * w), lambda bi, hi: (bi, 0, hi)),
            pl.BlockSpec((1, 1, c, w), lambda bi, hi: (bi, hi, 0, 0)),
            pl.BlockSpec((1, 1, c), lambda bi, hi: (bi, 0, 0)),
            pl.BlockSpec((c, c), lambda bi, hi: (0, 0)),
            pl.BlockSpec((1, c), lambda bi, hi: (0, 0)),
        ],
        out_specs=pl.BlockSpec((1, c, hb * w), lambda bi, hi: (bi, 0, hi)),
        out_shape=jax.ShapeDtypeStruct((b, c, n), jnp.float32),
    )(x.reshape(b, c, n), halo, y, Wg, bg[None, :])
    return out.reshape(b, c, h, w)


# speedup vs baseline: 2.1510x; 2.1510x over previous
import functools

import jax
import jax.numpy as jnp
from jax.experimental import pallas as pl


def _se_kernel(x_ref, w1t_ref, w2t_ref, y_ref, *, inv_n):
    hi = pl.program_id(1)
    nh = pl.num_programs(1)
    part = jnp.sum(x_ref[0], axis=(1, 2))[None, :]

    @pl.when(hi == 0)
    def _():
        y_ref[0] = part

    @pl.when(hi > 0)
    def _():
        y_ref[0] += part

    @pl.when(hi == nh - 1)
    def _():
        m = y_ref[0] * inv_n
        t = jnp.maximum(
            jnp.dot(m, w1t_ref[...], preferred_element_type=jnp.float32), 0.0)
        y_ref[0] = jax.nn.sigmoid(
            jnp.dot(t, w2t_ref[...], preferred_element_type=jnp.float32))


def _gcn_kernel(x_ref, xh_ref, y_ref, wg_ref, bg_ref, o_ref, *, hb, w):
    hi = pl.program_id(1)
    c = x_ref.shape[1]
    m = hb * w

    y = y_ref[0, 0]
    wgy = wg_ref[...] * y[None, :]
    zw = jnp.dot(wgy, x_ref[0].reshape(c, m),
                 preferred_element_type=jnp.float32)

    ri = jax.lax.broadcasted_iota(jnp.int32, (1, hb, w), 1) + hi * hb
    ci = jax.lax.broadcasted_iota(jnp.int32, (1, hb, w), 2)
    dinv = jax.lax.rsqrt(1.0 + (ri > 0).astype(jnp.float32)
                         + (ci > 0).astype(jnp.float32)).reshape(1, m)
    maskj = (ci > 0).astype(jnp.float32).reshape(1, m)
    z = zw * dinv

    zh = jnp.dot(wgy, xh_ref[0, 0], preferred_element_type=jnp.float32)
    cj = jax.lax.broadcasted_iota(jnp.int32, (1, w), 1)
    dinv_h = jax.lax.rsqrt(2.0 + (cj > 0).astype(jnp.float32))
    zh = jnp.where(hi == 0, 0.0, zh * dinv_h)

    zd = jnp.concatenate([zh, z[:, :m - w]], axis=1)
    zr = jnp.roll(z, 1, axis=1) * maskj

    res = (z + zd + zr) * dinv + bg_ref[0][:, None]
    o_ref[0] = res.reshape(c, hb, w)


@jax.jit
def kernel(x, W1, W2, Wg, bg):
    b, c, h, w = x.shape
    hb = 64
    nh = h // hb
    n = h * w

    y = pl.pallas_call(
        functools.partial(_se_kernel, inv_n=1.0 / n),
        grid=(b, nh),
        in_specs=[
            pl.BlockSpec((1, c, hb, w), lambda bi, hi: (bi, 0, hi, 0)),
            pl.BlockSpec((c, W1.shape[0]), lambda bi, hi: (0, 0)),
            pl.BlockSpec((W1.shape[0], c), lambda bi, hi: (0, 0)),
        ],
        out_specs=pl.BlockSpec((1, 1, c), lambda bi, hi: (bi, 0, 0)),
        out_shape=jax.ShapeDtypeStruct((b, 1, c), jnp.float32),
    )(x, W1.T, W2.T)

    halo = jnp.concatenate(
        [jnp.zeros((b, c, 1, w), x.dtype), x[:, :, hb - 1:h - 1:hb, :]],
        axis=2).transpose(0, 2, 1, 3)

    out = pl.pallas_call(
        functools.partial(_gcn_kernel, hb=hb, w=w),
        grid=(b, nh),
        in_specs=[
            pl.BlockSpec((1, c, hb, w), lambda bi, hi: (bi, 0, hi, 0)),
            pl.BlockSpec((1, 1, c, w), lambda bi, hi: (bi, hi, 0, 0)),
            pl.BlockSpec((1, 1, c), lambda bi, hi: (bi, 0, 0)),
            pl.BlockSpec((c, c), lambda bi, hi: (0, 0)),
            pl.BlockSpec((1, c), lambda bi, hi: (0, 0)),
        ],
        out_specs=pl.BlockSpec((1, c, hb, w), lambda bi, hi: (bi, 0, hi, 0)),
        out_shape=jax.ShapeDtypeStruct((b, c, h, w), jnp.float32),
    )(x, halo, y, Wg, bg[None, :])
    return out
